# Initial kernel scaffold; baseline (speedup 1.0000x reference)
import jax
import jax.numpy as jnp
from jax.experimental import pallas as pl


def kernel(x, p2e):
    raise NotImplementedError("write your pallas kernel here")


# speedup vs baseline: 8.2497x; 8.2497x over previous
import jax
import jax.numpy as jnp
from jax.experimental import pallas as pl
from jax.experimental.pallas import tpu as pltpu
from jax.experimental.pallas import tpu_sc as plsc

D_MODEL = 128
GATHER_WINDOW = 128


def kernel(x, p2e):
    shp = x.shape
    n = x.size
    flat = jnp.reshape(x, (1, n)).astype(jnp.int32)
    mesh = plsc.VectorSubcoreMesh(core_axis_name="core", subcore_axis_name="subcore")

    @pl.kernel(
        out_type=jax.ShapeDtypeStruct((n, D_MODEL), p2e.dtype),
        mesh=mesh,
    )
    def gather_kernel(table_hbm, idx_hbm, out_hbm):
        def body(i_vmem, o_vmem):
            pltpu.sync_copy(table_hbm.at[i_vmem.at[0]], o_vmem)

        pltpu.emit_pipeline(
            body,
            grid=(n // GATHER_WINDOW,),
            in_specs=[pl.BlockSpec((1, GATHER_WINDOW), index_map=lambda i: (0, i))],
            out_specs=[pl.BlockSpec((GATHER_WINDOW, D_MODEL), index_map=lambda i: (i, 0))],
            core_axis_name=("core", "subcore"),
            dimension_semantics=(pltpu.PARALLEL,),
        )(idx_hbm, out_hbm)

    out = gather_kernel(p2e, flat)
    return jnp.reshape(out, shp + (D_MODEL,))


# speedup vs baseline: 16.2907x; 1.9747x over previous
import jax
import jax.numpy as jnp
from jax import lax
from jax.experimental import pallas as pl
from jax.experimental.pallas import tpu as pltpu
from jax.experimental.pallas import tpu_sc as plsc

D_MODEL = 128
GATHER_WINDOW = 128


def kernel(x, p2e):
    shp = x.shape
    n = x.size
    flat = jnp.reshape(x, (1, n)).astype(jnp.int32)
    mesh = plsc.VectorSubcoreMesh(core_axis_name="core", subcore_axis_name="subcore")

    @pl.kernel(
        out_type=jax.ShapeDtypeStruct((n, D_MODEL), p2e.dtype),
        mesh=mesh,
        scratch_types=[pltpu.VMEM_SHARED(p2e.shape, p2e.dtype)],
    )
    def gather_kernel(table_hbm, idx_hbm, out_hbm, table_spmem):
        @pl.when(lax.axis_index("subcore") == 0)
        def _():
            pltpu.sync_copy(table_hbm, table_spmem)

        plsc.subcore_barrier()

        def body(i_vmem, o_vmem):
            pltpu.sync_copy(table_spmem.at[i_vmem.at[0]], o_vmem)

        pltpu.emit_pipeline(
            body,
            grid=(n // GATHER_WINDOW,),
            in_specs=[pl.BlockSpec((1, GATHER_WINDOW), index_map=lambda i: (0, i))],
            out_specs=[pl.BlockSpec((GATHER_WINDOW, D_MODEL), index_map=lambda i: (i, 0))],
            core_axis_name=("core", "subcore"),
            dimension_semantics=(pltpu.PARALLEL,),
        )(idx_hbm, out_hbm)

    out = gather_kernel(p2e, flat)
    return jnp.reshape(out, shp + (D_MODEL,))


# speedup vs baseline: 16.4387x; 1.0091x over previous
import jax
import jax.numpy as jnp
from jax import lax
from jax.experimental import pallas as pl
from jax.experimental.pallas import tpu as pltpu
from jax.experimental.pallas import tpu_sc as plsc

D_MODEL = 128
W = 128
WINDOWS = 200
PAIRS = WINDOWS // 2
NW = 32


def kernel(x, p2e):
    shp = x.shape
    n = x.size
    idx = jnp.reshape(x, (NW, WINDOWS, W)).astype(jnp.int32)
    mesh = plsc.VectorSubcoreMesh(core_axis_name="core", subcore_axis_name="subcore")

    @pl.kernel(
        out_type=jax.ShapeDtypeStruct((NW, WINDOWS, W, D_MODEL), p2e.dtype),
        mesh=mesh,
        scratch_types=[
            pltpu.VMEM_SHARED(p2e.shape, p2e.dtype),
            pltpu.VMEM((WINDOWS, W), jnp.int32),
            pltpu.VMEM((W, D_MODEL), p2e.dtype),
            pltpu.VMEM((W, D_MODEL), p2e.dtype),
            pltpu.SemaphoreType.DMA,
            pltpu.SemaphoreType.DMA,
        ],
    )
    def gather_kernel(
        table_hbm, idx_hbm, out_hbm, table_spmem, idx_v, buf0, buf1, sem0, sem1
    ):
        cid = lax.axis_index("core")
        sid = lax.axis_index("subcore")
        wid = sid * 2 + cid

        @pl.when(sid == 0)
        def _():
            pltpu.sync_copy(table_hbm, table_spmem)

        pltpu.sync_copy(idx_hbm.at[wid], idx_v)
        plsc.subcore_barrier()

        bufs = (buf0, buf1)
        sems = (sem0, sem1)

        def fill_and_send(s, b):
            pltpu.sync_copy(table_spmem.at[idx_v.at[s]], bufs[b])
            pltpu.async_copy(bufs[b], out_hbm.at[wid, s], sems[b])

        fill_and_send(0, 0)
        fill_and_send(1, 1)

        @pl.loop(1, WINDOWS // 2)
        def _(s2):
            for b in range(2):
                s = s2 * 2 + b
                pltpu.make_async_copy(bufs[b], out_hbm.at[wid, 0], sems[b]).wait()
                fill_and_send(s, b)

        for b in range(2):
            pltpu.make_async_copy(bufs[b], out_hbm.at[wid, 0], sems[b]).wait()

    out = gather_kernel(p2e, idx)
    return jnp.reshape(out, shp + (D_MODEL,))
